# baseline (device time: 10597 ns/iter reference)
import jax
import jax.numpy as jnp
from jax import lax
from jax.experimental import pallas as pl
from jax.experimental.pallas import tpu as pltpu

N_DEV = 16


def kernel(x):
    m_per, n = x.shape

    def body(x_hbm_ref, out_ref, xv_ref, gather_ref, load_sem,
             send_sems, recv_sems):
        my_pos = lax.axis_index("i")

        barrier_sem = pltpu.get_barrier_semaphore()
        for d in range(1, N_DEV):
            peer = lax.rem(my_pos + d, N_DEV)
            pl.semaphore_signal(
                barrier_sem, inc=1,
                device_id=(peer,), device_id_type=pl.DeviceIdType.MESH,
            )

        load = pltpu.make_async_copy(x_hbm_ref, xv_ref, load_sem)
        load.start()
        load.wait()

        xv = xv_ref[:, :]
        val = jnp.max(xv, axis=0)
        rows = lax.broadcasted_iota(jnp.int32, xv.shape, 0)
        masked = jnp.where(xv == val[None, :], rows, jnp.int32(1 << 30))
        lidx = jnp.min(masked, axis=0)
        gidx = (lidx + my_pos * m_per).astype(jnp.float32)
        gather_ref[N_DEV - 1, :, :] = jnp.stack([val, gidx], axis=0)

        pl.semaphore_wait(barrier_sem, N_DEV - 1)

        rdmas = []
        for d in range(1, N_DEV):
            peer = lax.rem(my_pos + d, N_DEV)
            rdma = pltpu.make_async_remote_copy(
                src_ref=gather_ref.at[N_DEV - 1],
                dst_ref=gather_ref.at[d - 1],
                send_sem=send_sems.at[d - 1],
                recv_sem=recv_sems.at[d - 1],
                device_id=(peer,),
                device_id_type=pl.DeviceIdType.MESH,
            )
            rdma.start()
            rdmas.append(rdma)

        for rdma in rdmas:
            rdma.wait_recv()

        vals = gather_ref[:, 0, :]
        idxs = gather_ref[:, 1, :]
        best = jnp.max(vals, axis=0)
        cand = jnp.where(vals == best[None, :], idxs, jnp.float32(3.0e7))
        bidx = jnp.min(cand, axis=0)
        out_ref[:, :] = jnp.stack([best, bidx], axis=0)

        for rdma in rdmas:
            rdma.wait_send()

    return pl.pallas_call(
        body,
        out_shape=jax.ShapeDtypeStruct((2, n), jnp.float32),
        in_specs=[pl.BlockSpec(memory_space=pl.ANY)],
        out_specs=pl.BlockSpec(memory_space=pltpu.VMEM),
        scratch_shapes=[
            pltpu.VMEM((m_per, n), jnp.float32),
            pltpu.VMEM((N_DEV, 2, n), jnp.float32),
            pltpu.SemaphoreType.DMA,
            pltpu.SemaphoreType.DMA((N_DEV - 1,)),
            pltpu.SemaphoreType.DMA((N_DEV - 1,)),
        ],
        compiler_params=pltpu.CompilerParams(collective_id=0),
    )(x)


# device time: 10460 ns/iter; 1.0131x vs baseline; 1.0131x over previous
import jax
import jax.numpy as jnp
from jax import lax
from jax.experimental import pallas as pl
from jax.experimental.pallas import tpu as pltpu

N_DEV = 16


def kernel(x):
    m_per, n = x.shape

    def body(x_hbm_ref, out_ref, xv_ref, gather_ref, load_sem,
             send_sems, recv_sems):
        my_pos = lax.axis_index("i")

        barrier_sem = pltpu.get_barrier_semaphore()
        pl.semaphore_wait(barrier_sem, 0)

        load = pltpu.make_async_copy(x_hbm_ref, xv_ref, load_sem)
        load.start()
        load.wait()

        xv = xv_ref[:, :]
        val = jnp.max(xv, axis=0)
        rows = lax.broadcasted_iota(jnp.int32, xv.shape, 0)
        masked = jnp.where(xv == val[None, :], rows, jnp.int32(1 << 30))
        lidx = jnp.min(masked, axis=0)
        gidx = (lidx + my_pos * m_per).astype(jnp.float32)
        gather_ref[N_DEV - 1, :, :] = jnp.stack([val, gidx], axis=0)

        rdmas = []
        for d in range(1, N_DEV):
            peer = lax.rem(my_pos + d, N_DEV)
            rdma = pltpu.make_async_remote_copy(
                src_ref=gather_ref.at[N_DEV - 1],
                dst_ref=gather_ref.at[d - 1],
                send_sem=send_sems.at[d - 1],
                recv_sem=recv_sems.at[d - 1],
                device_id=(peer,),
                device_id_type=pl.DeviceIdType.MESH,
            )
            rdma.start()
            rdmas.append(rdma)

        for rdma in rdmas:
            rdma.wait_recv()

        vals = gather_ref[:, 0, :]
        idxs = gather_ref[:, 1, :]
        best = jnp.max(vals, axis=0)
        cand = jnp.where(vals == best[None, :], idxs, jnp.float32(3.0e7))
        bidx = jnp.min(cand, axis=0)
        out_ref[:, :] = jnp.stack([best, bidx], axis=0)

        for rdma in rdmas:
            rdma.wait_send()

    return pl.pallas_call(
        body,
        out_shape=jax.ShapeDtypeStruct((2, n), jnp.float32),
        in_specs=[pl.BlockSpec(memory_space=pl.ANY)],
        out_specs=pl.BlockSpec(memory_space=pltpu.VMEM),
        scratch_shapes=[
            pltpu.VMEM((m_per, n), jnp.float32),
            pltpu.VMEM((N_DEV, 2, n), jnp.float32),
            pltpu.SemaphoreType.DMA,
            pltpu.SemaphoreType.DMA((N_DEV - 1,)),
            pltpu.SemaphoreType.DMA((N_DEV - 1,)),
        ],
        compiler_params=pltpu.CompilerParams(collective_id=0),
    )(x)
